# Initial kernel scaffold; baseline (speedup 1.0000x reference)
import jax
import jax.numpy as jnp
from jax.experimental import pallas as pl


def kernel(v_feat, r_feat, v2p_ind, r2p_ind, Wk0, Wv0, w0, Wk1, Wv1, w1):
    raise NotImplementedError("write your pallas kernel here")


# speedup vs baseline: 2.9082x; 2.9082x over previous
import functools

import jax
import jax.numpy as jnp
from jax import lax
from jax.experimental import pallas as pl
from jax.experimental.pallas import tpu as pltpu
from jax.experimental.pallas import tpu_sc as plsc

_NC, _NS, _LANES = 2, 16, 16
_NW = _NC * _NS

_B, _N = 2, 131072
_HWV = 512 * 512
_HWR = 256 * 256

_KB = 512
_KBC = 256
_CH = 24576
_ZROWS = 128
_NZ = 198
_CHP = _NZ * _ZROWS


def _dense_body(x_ref, wk_ref, wv_ref, w_ref, valt_ref, s_ref):
    x = x_ref[0]
    kt = lax.dot_general(x, wk_ref[...], (((0,), (1,)), ((), ())),
                         preferred_element_type=jnp.float32)
    vt = lax.dot_general(x, wv_ref[...], (((0,), (1,)), ((), ())),
                         preferred_element_type=jnp.float32)
    s = jnp.sum(jnp.tanh(kt) * w_ref[0][None, :], axis=1)
    valt_ref[0] = vt
    s_ref[...] = s[None, None, :]


def _dense_stage(x, wk, wv, w_row, tile):
    b, c, hw = x.shape
    grid = (b, hw // tile)
    return pl.pallas_call(
        _dense_body,
        grid=grid,
        in_specs=[
            pl.BlockSpec((1, c, tile), lambda i, j: (i, 0, j)),
            pl.BlockSpec((64, c), lambda i, j: (0, 0)),
            pl.BlockSpec((64, c), lambda i, j: (0, 0)),
            pl.BlockSpec((1, 64), lambda i, j: (0, 0)),
        ],
        out_specs=[
            pl.BlockSpec((1, tile, 64), lambda i, j: (i, j, 0)),
            pl.BlockSpec((1, 1, tile), lambda i, j, nj=hw // tile:
                         (i * nj + j, 0, 0)),
        ],
        out_shape=[
            jax.ShapeDtypeStruct((b, hw, 64), jnp.float32),
            jax.ShapeDtypeStruct((b * (hw // tile), 1, tile), jnp.float32),
        ],
    )(x, wk, wv, w_row)


def _sc_gather_body(val0_hbm, val1_hbm, s0_hbm, s1_hbm, gv_hbm, gr_hbm,
                    fused_hbm,
                    idxv, idxr, rows0, rows1, s0b, s1b, coefa, fblk, sem):
    wid = lax.axis_index("s") * _NC + lax.axis_index("c")
    npts = (_B * _N) // _NW
    nblk = npts // _KB
    base = wid * npts

    def blk(i, carry):
        off = pl.multiple_of(base + i * _KB, 512)
        pltpu.sync_copy(gv_hbm.at[pl.ds(off, _KB)], idxv)
        pltpu.sync_copy(gr_hbm.at[pl.ds(off, _KB)], idxr)
        descs = []
        for j in range(_KB // 128):
            sl = pl.ds(j * 128, 128)
            descs.append(pltpu.async_copy(val0_hbm.at[idxv.at[sl]],
                                          rows0.at[sl], sem))
            descs.append(pltpu.async_copy(val1_hbm.at[idxr.at[sl]],
                                          rows1.at[sl], sem))
            descs.append(pltpu.async_copy(s0_hbm.at[idxv.at[sl]],
                                          s0b.at[sl], sem))
            descs.append(pltpu.async_copy(s1_hbm.at[idxr.at[sl]],
                                          s1b.at[sl], sem))
        for d in descs:
            d.wait()

        def coef(g, c2):
            sl = pl.ds(g * _LANES, _LANES)
            e = jnp.exp(s1b[sl] - s0b[sl])
            coefa[sl] = 1.0 / (1.0 + e)
            return c2
        lax.fori_loop(0, _KB // _LANES, coef, 0)

        def pt(p, c3):
            a = coefa[pl.ds(p, _LANES)][0]
            av = jnp.full((_LANES,), a, jnp.float32)
            for c4 in range(4):
                sl = pl.ds(c4 * _LANES, _LANES)
                v0 = rows0[p, sl]
                v1 = rows1[p, sl]
                fblk[p, sl] = v1 + v1 + (v0 - v1) * av
            return c3
        lax.fori_loop(0, _KB, pt, 0)

        pltpu.sync_copy(fblk, fused_hbm.at[pl.ds(off, _KB)])
        return carry

    lax.fori_loop(0, nblk, blk, 0)


def _sc_gather(val0_tbl, val1_tbl, s0_tbl, s1_tbl, gv1, gr1):
    mesh = plsc.VectorSubcoreMesh(core_axis_name="c", subcore_axis_name="s")
    fn = functools.partial(
        pl.kernel,
        out_type=jax.ShapeDtypeStruct((_B * _N, 64), jnp.float32),
        mesh=mesh,
        compiler_params=pltpu.CompilerParams(use_tc_tiling_on_sc=False),
        scratch_types=[
            pltpu.VMEM((_KB,), jnp.int32),
            pltpu.VMEM((_KB,), jnp.int32),
            pltpu.VMEM((_KB, 64), jnp.float32),
            pltpu.VMEM((_KB, 64), jnp.float32),
            pltpu.VMEM((_KB,), jnp.float32),
            pltpu.VMEM((_KB,), jnp.float32),
            pltpu.VMEM((_KB + _LANES,), jnp.float32),
            pltpu.VMEM((_KB, 64), jnp.float32),
            pltpu.SemaphoreType.DMA,
        ],
    )(_sc_gather_body)
    return fn(val0_tbl, val1_tbl, s0_tbl, s1_tbl, gv1, gr1)


def _sc_scatter_body(fused_hbm, oidx_hbm, outt_hbm,
                     oidxb, adj0, adj1, fblk, zbuf, chunk):
    b = lax.axis_index("c")
    s = lax.axis_index("s")
    adjs = [adj0, adj1]

    def zr(r, c0):
        for c4 in range(4):
            zbuf[r, pl.ds(c4 * _LANES, _LANES)] = jnp.zeros((_LANES,),
                                                            jnp.float32)
        return c0
    lax.fori_loop(0, _ZROWS, zr, 0)

    nblk = (_N // _NS) // _KBC
    for c in range(3):
        cbase = c * _CH
        csize = _CH if c < 2 else _HWR - 2 * _CH

        for k in range(13):
            m = k * _NS + s

            @pl.when(m < _NZ)
            def _():
                zoff = pl.multiple_of(m * _ZROWS, _ZROWS)
                pltpu.sync_copy(zbuf, chunk.at[pl.ds(zoff, _ZROWS)])
        plsc.subcore_barrier()

        def blk(i, c1):
            off = pl.multiple_of(
                b * _N + s * (_N // _NS) + i * _KBC, _KBC)
            pltpu.sync_copy(oidx_hbm.at[pl.ds(off, _KBC)], oidxb)
            pltpu.sync_copy(fused_hbm.at[pl.ds(off, _KBC)], fblk)
            for j in range(_KBC // 128):
                for g in range(128 // _LANES):
                    sl = pl.ds(g * _LANES, _LANES)
                    oi = oidxb[pl.ds(j * 128 + g * _LANES, _LANES)]
                    rel = oi - cbase
                    msk = jnp.logical_and(rel >= 0, rel < csize)
                    trash = _CH + (oi & 127)
                    adjs[j][sl] = jnp.where(msk, rel, trash)
            for j in range(_KBC // 128):
                pltpu.sync_copy(fblk.at[pl.ds(j * 128, 128)],
                                chunk.at[adjs[j]], add=True)
            return c1
        lax.fori_loop(0, nblk, blk, 0)
        plsc.subcore_barrier()

        share = csize // _NS
        pltpu.sync_copy(
            chunk.at[pl.ds(pl.multiple_of(s * share, share), share)],
            outt_hbm.at[pl.ds(
                pl.multiple_of(b * _HWR + cbase + s * share, share), share)])
        plsc.subcore_barrier()


def _sc_scatter(fused, oidx1):
    mesh = plsc.VectorSubcoreMesh(core_axis_name="c", subcore_axis_name="s")
    fn = functools.partial(
        pl.kernel,
        out_type=jax.ShapeDtypeStruct((_B * _HWR, 64), jnp.float32),
        mesh=mesh,
        compiler_params=pltpu.CompilerParams(use_tc_tiling_on_sc=False),
        scratch_types=[
            pltpu.VMEM((_KBC,), jnp.int32),
            pltpu.VMEM((128,), jnp.int32),
            pltpu.VMEM((128,), jnp.int32),
            pltpu.VMEM((_KBC, 64), jnp.float32),
            pltpu.VMEM((_ZROWS, 64), jnp.float32),
            pltpu.VMEM_SHARED((_CHP, 64), jnp.float32),
        ],
    )(_sc_scatter_body)
    return fn(fused, oidx1)


def _trans_body(x_ref, o_ref):
    x = x_ref[0]
    r = lax.broadcasted_iota(jnp.int32, (64, 64), 0)
    cc = lax.broadcasted_iota(jnp.int32, (64, 64), 1)
    eye = (r == cc).astype(jnp.float32)
    o_ref[0] = lax.dot_general(eye, x, (((1,), (1,)), ((), ())),
                               preferred_element_type=jnp.float32)


def _trans_stage(x, tile):
    b, hw, c = x.shape
    grid = (b, hw // tile)
    return pl.pallas_call(
        _trans_body,
        grid=grid,
        in_specs=[pl.BlockSpec((1, tile, c), lambda i, j: (i, j, 0))],
        out_specs=pl.BlockSpec((1, c, tile), lambda i, j: (i, 0, j)),
        out_shape=jax.ShapeDtypeStruct((b, c, hw), jnp.float32),
    )(x)


def kernel(v_feat, r_feat, v2p_ind, r2p_ind, Wk0, Wv0, w0, Wk1, Wv1, w1):
    B = v_feat.shape[0]

    val0t, s0 = _dense_stage(v_feat.reshape(B, 64, _HWV), Wk0, Wv0,
                             w0.reshape(1, 64), 512)
    val1t, s1 = _dense_stage(r_feat.reshape(B, 20, _HWR), Wk1, Wv1,
                             w1.reshape(1, 64), 512)

    boffv = (jnp.arange(B, dtype=jnp.int32) * _HWV)[:, None]
    boffr = (jnp.arange(B, dtype=jnp.int32) * _HWR)[:, None]
    gv = (v2p_ind[..., 0] * 512 + v2p_ind[..., 1] + boffv).reshape(-1)
    gr = (r2p_ind[..., 0] * 256 + r2p_ind[..., 1] + boffr).reshape(-1)
    oidx = (r2p_ind[..., 0] * 256 + r2p_ind[..., 1]).reshape(-1)

    fused = _sc_gather(val0t.reshape(_B * _HWV, 64),
                       val1t.reshape(_B * _HWR, 64),
                       s0.reshape(_B * _HWV), s1.reshape(_B * _HWR), gv, gr)

    outt = _sc_scatter(fused, oidx)

    out = _trans_stage(outt.reshape(B, _HWR, 64), 512)
    return out.reshape(B, 64, 256, 256)


# speedup vs baseline: 4.5304x; 1.5578x over previous
import functools

import jax
import jax.numpy as jnp
from jax import lax
from jax.experimental import pallas as pl
from jax.experimental.pallas import tpu as pltpu
from jax.experimental.pallas import tpu_sc as plsc

_NC, _NS, _LANES = 2, 16, 16
_NW = _NC * _NS

_B, _N = 2, 131072
_HWV = 512 * 512
_HWR = 256 * 256

_KB = 512
_KBC = 256
_CH = 24576
_ZROWS = 128
_NZ = 198
_CHP = _NZ * _ZROWS


def _dense_body(x_ref, wk_ref, wv_ref, w_ref, valt_ref, s_ref):
    x = x_ref[0]
    k = lax.dot_general(wk_ref[...], x, (((1,), (0,)), ((), ())),
                        preferred_element_type=jnp.float32)
    s = lax.dot_general(w_ref[...], jnp.tanh(k), (((1,), (0,)), ((), ())),
                        preferred_element_type=jnp.float32)
    vt = lax.dot_general(x, wv_ref[...], (((0,), (1,)), ((), ())),
                         preferred_element_type=jnp.float32)
    valt_ref[0] = vt
    s_ref[...] = s[:, None, :]


def _dense_stage(x, wk, wv, w_row, tile):
    b, c, hw = x.shape
    grid = (b, hw // tile)
    return pl.pallas_call(
        _dense_body,
        grid=grid,
        in_specs=[
            pl.BlockSpec((1, c, tile), lambda i, j: (i, 0, j)),
            pl.BlockSpec((64, c), lambda i, j: (0, 0)),
            pl.BlockSpec((64, c), lambda i, j: (0, 0)),
            pl.BlockSpec((1, 64), lambda i, j: (0, 0)),
        ],
        out_specs=[
            pl.BlockSpec((1, tile, 64), lambda i, j: (i, j, 0)),
            pl.BlockSpec((1, 1, tile), lambda i, j, nj=hw // tile:
                         (i * nj + j, 0, 0)),
        ],
        out_shape=[
            jax.ShapeDtypeStruct((b, hw, 64), jnp.float32),
            jax.ShapeDtypeStruct((b * (hw // tile), 1, tile), jnp.float32),
        ],
    )(x, wk, wv, w_row)


def _sc_gather_body(val0_hbm, val1_hbm, s0_hbm, s1_hbm, gv_hbm, gr_hbm,
                    fused_hbm,
                    idxv, idxr, rows0, rows1, s0b, s1b, coefa, fblk, sem):
    wid = lax.axis_index("s") * _NC + lax.axis_index("c")
    npts = (_B * _N) // _NW
    nblk = npts // _KB
    base = wid * npts

    def blk(i, carry):
        off = pl.multiple_of(base + i * _KB, 512)
        pltpu.sync_copy(gv_hbm.at[pl.ds(off, _KB)], idxv)
        pltpu.sync_copy(gr_hbm.at[pl.ds(off, _KB)], idxr)
        descs = []
        for j in range(_KB // 128):
            sl = pl.ds(j * 128, 128)
            descs.append(pltpu.async_copy(val0_hbm.at[idxv.at[sl]],
                                          rows0.at[sl], sem))
            descs.append(pltpu.async_copy(val1_hbm.at[idxr.at[sl]],
                                          rows1.at[sl], sem))
            descs.append(pltpu.async_copy(s0_hbm.at[idxv.at[sl]],
                                          s0b.at[sl], sem))
            descs.append(pltpu.async_copy(s1_hbm.at[idxr.at[sl]],
                                          s1b.at[sl], sem))
        for d in descs:
            d.wait()

        def coef(g, c2):
            sl = pl.ds(g * _LANES, _LANES)
            e = jnp.exp(s1b[sl] - s0b[sl])
            coefa[sl] = 1.0 / (1.0 + e)
            return c2
        lax.fori_loop(0, _KB // _LANES, coef, 0)

        def pt(p, c3):
            a = coefa[pl.ds(p, _LANES)][0]
            av = jnp.full((_LANES,), a, jnp.float32)
            for c4 in range(4):
                sl = pl.ds(c4 * _LANES, _LANES)
                v0 = rows0[p, sl]
                v1 = rows1[p, sl]
                fblk[p, sl] = v1 + v1 + (v0 - v1) * av
            return c3
        lax.fori_loop(0, _KB, pt, 0)

        pltpu.sync_copy(fblk, fused_hbm.at[pl.ds(off, _KB)])
        return carry

    lax.fori_loop(0, nblk, blk, 0)


def _sc_gather(val0_tbl, val1_tbl, s0_tbl, s1_tbl, gv1, gr1):
    mesh = plsc.VectorSubcoreMesh(core_axis_name="c", subcore_axis_name="s")
    fn = functools.partial(
        pl.kernel,
        out_type=jax.ShapeDtypeStruct((_B * _N, 64), jnp.float32),
        mesh=mesh,
        compiler_params=pltpu.CompilerParams(use_tc_tiling_on_sc=False),
        scratch_types=[
            pltpu.VMEM((_KB,), jnp.int32),
            pltpu.VMEM((_KB,), jnp.int32),
            pltpu.VMEM((_KB, 64), jnp.float32),
            pltpu.VMEM((_KB, 64), jnp.float32),
            pltpu.VMEM((_KB,), jnp.float32),
            pltpu.VMEM((_KB,), jnp.float32),
            pltpu.VMEM((_KB + _LANES,), jnp.float32),
            pltpu.VMEM((_KB, 64), jnp.float32),
            pltpu.SemaphoreType.DMA,
        ],
    )(_sc_gather_body)
    return fn(val0_tbl, val1_tbl, s0_tbl, s1_tbl, gv1, gr1)


def _sc_scatter_body(fused_hbm, oidx_hbm, outt_hbm,
                     oidxb, adj0, adj1, fblk, zbuf, chunk):
    b = lax.axis_index("c")
    s = lax.axis_index("s")
    adjs = [adj0, adj1]

    def zr(r, c0):
        for c4 in range(4):
            zbuf[r, pl.ds(c4 * _LANES, _LANES)] = jnp.zeros((_LANES,),
                                                            jnp.float32)
        return c0
    lax.fori_loop(0, _ZROWS, zr, 0)

    nblk = (_N // _NS) // _KBC
    for c in range(3):
        cbase = c * _CH
        csize = _CH if c < 2 else _HWR - 2 * _CH

        for k in range(13):
            m = k * _NS + s

            @pl.when(m < _NZ)
            def _():
                zoff = pl.multiple_of(m * _ZROWS, _ZROWS)
                pltpu.sync_copy(zbuf, chunk.at[pl.ds(zoff, _ZROWS)])
        plsc.subcore_barrier()

        def blk(i, c1):
            off = pl.multiple_of(
                b * _N + s * (_N // _NS) + i * _KBC, _KBC)
            pltpu.sync_copy(oidx_hbm.at[pl.ds(off, _KBC)], oidxb)
            pltpu.sync_copy(fused_hbm.at[pl.ds(off, _KBC)], fblk)
            for j in range(_KBC // 128):
                for g in range(128 // _LANES):
                    sl = pl.ds(g * _LANES, _LANES)
                    oi = oidxb[pl.ds(j * 128 + g * _LANES, _LANES)]
                    rel = oi - cbase
                    msk = jnp.logical_and(rel >= 0, rel < csize)
                    trash = _CH + (oi & 127)
                    adjs[j][sl] = jnp.where(msk, rel, trash)
            for j in range(_KBC // 128):
                pltpu.sync_copy(fblk.at[pl.ds(j * 128, 128)],
                                chunk.at[adjs[j]], add=True)
            return c1
        lax.fori_loop(0, nblk, blk, 0)
        plsc.subcore_barrier()

        share = csize // _NS
        pltpu.sync_copy(
            chunk.at[pl.ds(pl.multiple_of(s * share, share), share)],
            outt_hbm.at[pl.ds(
                pl.multiple_of(b * _HWR + cbase + s * share, share), share)])
        plsc.subcore_barrier()


def _sc_scatter(fused, oidx1):
    mesh = plsc.VectorSubcoreMesh(core_axis_name="c", subcore_axis_name="s")
    fn = functools.partial(
        pl.kernel,
        out_type=jax.ShapeDtypeStruct((_B * _HWR, 64), jnp.float32),
        mesh=mesh,
        compiler_params=pltpu.CompilerParams(use_tc_tiling_on_sc=False),
        scratch_types=[
            pltpu.VMEM((_KBC,), jnp.int32),
            pltpu.VMEM((128,), jnp.int32),
            pltpu.VMEM((128,), jnp.int32),
            pltpu.VMEM((_KBC, 64), jnp.float32),
            pltpu.VMEM((_ZROWS, 64), jnp.float32),
            pltpu.VMEM_SHARED((_CHP, 64), jnp.float32),
        ],
    )(_sc_scatter_body)
    return fn(fused, oidx1)


def _trans_body(x_ref, o_ref):
    x = x_ref[0]
    r = lax.broadcasted_iota(jnp.int32, (64, 64), 0)
    cc = lax.broadcasted_iota(jnp.int32, (64, 64), 1)
    eye = (r == cc).astype(jnp.float32)
    o_ref[0] = lax.dot_general(eye, x, (((1,), (1,)), ((), ())),
                               preferred_element_type=jnp.float32)


def _trans_stage(x, tile):
    b, hw, c = x.shape
    grid = (b, hw // tile)
    return pl.pallas_call(
        _trans_body,
        grid=grid,
        in_specs=[pl.BlockSpec((1, tile, c), lambda i, j: (i, j, 0))],
        out_specs=pl.BlockSpec((1, c, tile), lambda i, j: (i, 0, j)),
        out_shape=jax.ShapeDtypeStruct((b, c, hw), jnp.float32),
    )(x)


def kernel(v_feat, r_feat, v2p_ind, r2p_ind, Wk0, Wv0, w0, Wk1, Wv1, w1):
    B = v_feat.shape[0]

    val0t, s0 = _dense_stage(v_feat.reshape(B, 64, _HWV), Wk0, Wv0,
                             w0.reshape(1, 64), 2048)
    val1t, s1 = _dense_stage(r_feat.reshape(B, 20, _HWR), Wk1, Wv1,
                             w1.reshape(1, 64), 2048)

    boffv = (jnp.arange(B, dtype=jnp.int32) * _HWV)[:, None]
    boffr = (jnp.arange(B, dtype=jnp.int32) * _HWR)[:, None]
    gv = (v2p_ind[..., 0] * 512 + v2p_ind[..., 1] + boffv).reshape(-1)
    gr = (r2p_ind[..., 0] * 256 + r2p_ind[..., 1] + boffr).reshape(-1)
    oidx = (r2p_ind[..., 0] * 256 + r2p_ind[..., 1]).reshape(-1)

    fused = _sc_gather(val0t.reshape(_B * _HWV, 64),
                       val1t.reshape(_B * _HWR, 64),
                       s0.reshape(_B * _HWV), s1.reshape(_B * _HWR), gv, gr)

    outt = _sc_scatter(fused, oidx)

    out = _trans_stage(outt.reshape(B, _HWR, 64), 2048)
    return out.reshape(B, 64, 256, 256)
